# baseline (speedup 1.0000x reference)
import functools

import jax
import jax.numpy as jnp
from jax import lax
from jax.experimental import pallas as pl
from jax.experimental.pallas import tpu as pltpu
from jax.experimental.pallas import tpu_sc as plsc

B = 16
P = 1024
K = 20
N = B * P
M = K * N

_BLKR = 1024
_BLKE = 256


def _knn_uv_body(x_blk_ref, x_all_ref, wd_ref, b1_ref, wb_ref,
                 idx_ref, u_ref, v_ref):
    b = pl.program_id(0)
    i = pl.program_id(1)
    xb = x_blk_ref[0]
    xa = x_all_ref[0]
    dots = lax.dot_general(xb, xa, (((1,), (1,)), ((), ())),
                           preferred_element_type=jnp.float32)
    sqb = jnp.sum(xb * xb, axis=1, keepdims=True)
    sqa = jnp.sum(xa * xa, axis=1, keepdims=True)
    d2 = jnp.maximum(sqb - 2.0 * dots + sqa.reshape(1, P), 0.0)
    rows = lax.broadcasted_iota(jnp.int32, (_BLKR, P), 0) + i * _BLKR
    cols = lax.broadcasted_iota(jnp.int32, (_BLKR, P), 1)
    ki = lax.bitcast_convert_type(d2, jnp.int32)
    ki = jnp.bitwise_or(jnp.bitwise_and(ki, jnp.int32(~1023)), cols)
    ki = jnp.where(rows == cols, jnp.int32(0x7FFFFFFF), ki)
    base = b * P
    H = _BLKR // 2
    IMIN = jnp.int32(-2147483648)
    kh = (ki[:H], ki[H:])
    mm = [jnp.full((H, 1), IMIN, jnp.int32), jnp.full((H, 1), IMIN, jnp.int32)]
    for t in range(K):
        for s in range(2):
            mn = jnp.min(kh[s] - mm[s], axis=1, keepdims=True)
            m = mn + mm[s]
            at = jnp.bitwise_and(m, jnp.int32(1023)) + base
            idx_ref[0, pl.ds(s * H, H), t] = at[:, 0]
            mm[s] = m + (IMIN + 1)
    u_ref[0] = (jnp.dot(xb, wd_ref[...], preferred_element_type=jnp.float32)
                + b1_ref[...])
    if v_ref is not None:
        v_ref[0] = jnp.dot(xb, wb_ref[...],
                           preferred_element_type=jnp.float32)


def _knn_u_body(x_blk_ref, x_all_ref, wd_ref, b1_ref, idx_ref, u_ref):
    _knn_uv_body(x_blk_ref, x_all_ref, wd_ref, b1_ref, None, idx_ref,
                 u_ref, None)


def _knn_uv(x, wd, b1, wb):
    Bn, _, D = x.shape
    C = wd.shape[1]
    grid = (Bn, P // _BLKR)
    return pl.pallas_call(
        _knn_uv_body,
        grid=grid,
        in_specs=[
            pl.BlockSpec((1, _BLKR, D), lambda b, i: (b, i, 0)),
            pl.BlockSpec((1, P, D), lambda b, i: (b, 0, 0)),
            pl.BlockSpec((D, C), lambda b, i: (0, 0)),
            pl.BlockSpec((1, C), lambda b, i: (0, 0)),
            pl.BlockSpec((D, C), lambda b, i: (0, 0)),
        ],
        out_specs=[
            pl.BlockSpec((1, _BLKR, K), lambda b, i: (b, i, 0)),
            pl.BlockSpec((1, _BLKR, C), lambda b, i: (b, i, 0)),
            pl.BlockSpec((1, _BLKR, C), lambda b, i: (b, i, 0)),
        ],
        out_shape=[
            jax.ShapeDtypeStruct((Bn, P, K), jnp.int32),
            jax.ShapeDtypeStruct((Bn, P, C), jnp.float32),
            jax.ShapeDtypeStruct((Bn, P, C), jnp.float32),
        ],
    )(x, x, wd, b1, wb)


def _knn_u(x, wd, b1):
    Bn, _, D = x.shape
    C = wd.shape[1]
    grid = (Bn, P // _BLKR)
    return pl.pallas_call(
        _knn_u_body,
        grid=grid,
        in_specs=[
            pl.BlockSpec((1, _BLKR, D), lambda b, i: (b, i, 0)),
            pl.BlockSpec((1, P, D), lambda b, i: (b, 0, 0)),
            pl.BlockSpec((D, C), lambda b, i: (0, 0)),
            pl.BlockSpec((1, C), lambda b, i: (0, 0)),
        ],
        out_specs=[
            pl.BlockSpec((1, _BLKR, K), lambda b, i: (b, i, 0)),
            pl.BlockSpec((1, _BLKR, C), lambda b, i: (b, i, 0)),
        ],
        out_shape=[
            jax.ShapeDtypeStruct((Bn, P, K), jnp.int32),
            jax.ShapeDtypeStruct((Bn, P, C), jnp.float32),
        ],
    )(x, x, wd, b1)


_SC_INFO = plsc.get_sparse_core_info()
_NC = _SC_INFO.num_cores
_NS = _SC_INFO.num_subcores
_NW = _NC * _NS
_G = 80


@functools.lru_cache(maxsize=None)
def _make_sc_gather(C, Mn):
    per_w = Mn // _NW
    iters = per_w // _G
    mesh = plsc.VectorSubcoreMesh(core_axis_name="c", subcore_axis_name="s")

    @functools.partial(
        pl.kernel, mesh=mesh,
        out_type=jax.ShapeDtypeStruct((Mn, C), jnp.float32),
        scratch_types=[
            pltpu.VMEM((per_w,), jnp.int32),
            pltpu.VMEM((4, _G, C), jnp.float32),
            [pltpu.SemaphoreType.DMA] * 4,
            [pltpu.SemaphoreType.DMA] * 4,
        ],
    )
    def sc_gather(table_hbm, idx_hbm, out_hbm, idxall, rows, gsems, wsems):
        wid = lax.axis_index("s") * _NC + lax.axis_index("c")
        base = wid * per_w
        pltpu.sync_copy(idx_hbm.at[pl.ds(base, per_w)], idxall)

        def start_gather(g, s):
            pltpu.async_copy(
                table_hbm.at[idxall.at[pl.ds(g * _G, _G)]],
                rows.at[s], gsems[s])

        start_gather(0, 0)
        start_gather(1, 1)

        def body(gg, carry):
            for s in range(4):
                g = gg * 4 + s
                s2 = (s + 2) % 4
                pltpu.make_async_copy(
                    table_hbm.at[idxall.at[pl.ds(g * _G, _G)]],
                    rows.at[s], gsems[s]).wait()
                pltpu.async_copy(
                    rows.at[s], out_hbm.at[pl.ds(base + g * _G, _G)],
                    wsems[s])

                @pl.when(jnp.logical_and(g >= 2, g + 2 < iters))
                def _():
                    pltpu.make_async_copy(
                        rows.at[s2], out_hbm.at[pl.ds(base, _G)],
                        wsems[s2]).wait()

                @pl.when(g + 2 < iters)
                def _():
                    start_gather(g + 2, s2)
            return carry

        lax.fori_loop(0, iters // 4, body, 0)
        for s in range(4):
            pltpu.make_async_copy(
                rows.at[s], out_hbm.at[pl.ds(base, _G)], wsems[s]).wait()

    return sc_gather


def _edge_body_full(u_ref, vg_ref, w2_ref, b2_ref, out_ref, pooled_ref):
    i = pl.program_id(1)
    u = u_ref[0]
    acc = jnp.full((u.shape[0], w2_ref.shape[-1]), -1e30, jnp.float32)
    for t in range(K):
        h = jnp.maximum(u + vg_ref[0, t], 0.0)
        acc = jnp.maximum(
            acc, jnp.dot(h, w2_ref[...], preferred_element_type=jnp.float32))
    y = acc + b2_ref[...]
    out_ref[0] = y
    pm = jnp.max(y, axis=0, keepdims=True)

    @pl.when(i == 0)
    def _():
        pooled_ref[0] = pm

    @pl.when(i > 0)
    def _():
        pooled_ref[0] = jnp.maximum(pooled_ref[0], pm)


def _edge_body_pool(u_ref, vg_ref, w2_ref, b2_ref, pooled_ref):
    i = pl.program_id(1)
    u = u_ref[0]
    acc = jnp.full((u.shape[0], w2_ref.shape[-1]), -1e30, jnp.float32)
    for t in range(K):
        h = jnp.maximum(u + vg_ref[0, t], 0.0)
        acc = jnp.maximum(
            acc, jnp.dot(h, w2_ref[...], preferred_element_type=jnp.float32))
    y = acc + b2_ref[...]
    pm = jnp.max(y, axis=0, keepdims=True)

    @pl.when(i == 0)
    def _():
        pooled_ref[0] = pm

    @pl.when(i > 0)
    def _():
        pooled_ref[0] = jnp.maximum(pooled_ref[0], pm)


def _edge(u, vg, w2, b2, with_out):
    Bn, Ci = u.shape[0], u.shape[-1]
    Co = w2.shape[1]
    grid = (Bn, P // _BLKE)
    in_specs = [
        pl.BlockSpec((1, _BLKE, Ci), lambda b, i: (b, i, 0)),
        pl.BlockSpec((1, K, _BLKE, Ci), lambda b, i: (b, 0, i, 0)),
        pl.BlockSpec((Ci, Co), lambda b, i: (0, 0)),
        pl.BlockSpec((1, Co), lambda b, i: (0, 0)),
    ]
    pooled_spec = pl.BlockSpec((1, 1, Co), lambda b, i: (b, 0, 0))
    if with_out:
        return pl.pallas_call(
            _edge_body_full,
            grid=grid,
            in_specs=in_specs,
            out_specs=[pl.BlockSpec((1, _BLKE, Co), lambda b, i: (b, i, 0)),
                       pooled_spec],
            out_shape=[jax.ShapeDtypeStruct((Bn, P, Co), jnp.float32),
                       jax.ShapeDtypeStruct((Bn, 1, Co), jnp.float32)],
        )(u, vg, w2, b2)
    return pl.pallas_call(
        _edge_body_pool,
        grid=grid,
        in_specs=in_specs,
        out_specs=[pooled_spec],
        out_shape=[jax.ShapeDtypeStruct((Bn, 1, Co), jnp.float32)],
    )(u, vg, w2, b2)


def _edge_body_pool_x(u_ref, xg_ref, wb_ref, w2_ref, b2_ref, pooled_ref):
    i = pl.program_id(1)
    u = u_ref[0]
    acc = jnp.full((u.shape[0], w2_ref.shape[-1]), -1e30, jnp.float32)
    for t in range(K):
        vj = jnp.dot(xg_ref[0, t], wb_ref[...],
                     preferred_element_type=jnp.float32)
        h = jnp.maximum(u + vj, 0.0)
        acc = jnp.maximum(
            acc, jnp.dot(h, w2_ref[...], preferred_element_type=jnp.float32))
    y = acc + b2_ref[...]
    pm = jnp.max(y, axis=0, keepdims=True)

    @pl.when(i == 0)
    def _():
        pooled_ref[0] = pm

    @pl.when(i > 0)
    def _():
        pooled_ref[0] = jnp.maximum(pooled_ref[0], pm)


def _edge_x(u, xg, wb, w2, b2):
    Bn, Ci = u.shape[0], u.shape[-1]
    D = xg.shape[-1]
    Co = w2.shape[1]
    grid = (Bn, P // _BLKE)
    return pl.pallas_call(
        _edge_body_pool_x,
        grid=grid,
        in_specs=[
            pl.BlockSpec((1, _BLKE, Ci), lambda b, i: (b, i, 0)),
            pl.BlockSpec((1, K, _BLKE, D), lambda b, i: (b, 0, i, 0)),
            pl.BlockSpec((D, Co), lambda b, i: (0, 0)),
            pl.BlockSpec((Ci, Co), lambda b, i: (0, 0)),
            pl.BlockSpec((1, Co), lambda b, i: (0, 0)),
        ],
        out_specs=[pl.BlockSpec((1, 1, Co), lambda b, i: (b, 0, 0))],
        out_shape=[jax.ShapeDtypeStruct((Bn, 1, Co), jnp.float32)],
    )(u, xg, wb, w2, b2)


def _head_body(p1_ref, p2_ref, p3_ref, p4_ref, wa_ref, wb_ref, wc_ref, wd_ref,
               l1b_ref, bng_ref, bnb_ref, l2w_ref, l2b_ref, out_ref):
    h = (jnp.dot(p1_ref[...], wa_ref[...], preferred_element_type=jnp.float32)
         + jnp.dot(p2_ref[...], wb_ref[...], preferred_element_type=jnp.float32)
         + jnp.dot(p3_ref[...], wc_ref[...], preferred_element_type=jnp.float32)
         + jnp.dot(p4_ref[...], wd_ref[...], preferred_element_type=jnp.float32)
         + l1b_ref[...])
    mu = jnp.mean(h, axis=0, keepdims=True)
    var = jnp.mean((h - mu) * (h - mu), axis=0, keepdims=True)
    hn = (h - mu) / jnp.sqrt(var + 1e-5) * bng_ref[...] + bnb_ref[...]
    hr = jnp.maximum(hn, 0.0)
    logits = (jnp.dot(hr, l2w_ref[...], preferred_element_type=jnp.float32)
              + l2b_ref[...])
    mx = jnp.max(logits, axis=1, keepdims=True)
    z = logits - mx
    lse = jnp.log(jnp.sum(jnp.exp(z), axis=1, keepdims=True))
    out_ref[...] = z - lse


def _head(p1, p2, p3, p4, l1w, l1b, bng, bnb, l2w, l2b):
    wa, wb, wc, wd = l1w[:64], l1w[64:128], l1w[128:256], l1w[256:]
    args = [p1, p2, p3, p4, wa, wb, wc, wd, l1b.reshape(1, -1),
            bng.reshape(1, -1), bnb.reshape(1, -1), l2w, l2b.reshape(1, -1)]
    return pl.pallas_call(
        _head_body,
        out_shape=jax.ShapeDtypeStruct((B, 40), jnp.float32),
    )(*args)


def _half_layer(x, wd, wb, b1, w2, b2, with_out, gather_x):
    Bn, _, D = x.shape
    Nn = Bn * P
    Mn = K * Nn
    C = wd.shape[1]
    if gather_x:
        idxi, u = _knn_u(x, wd, b1)
        gidx = jnp.transpose(idxi, (0, 2, 1)).reshape(Mn)
        xg = _make_sc_gather(D, Mn)(x.reshape(Nn, D), gidx)
        return _edge_x(u, xg.reshape(Bn, K, P, D), wb, w2, b2)
    idxi, u, v = _knn_uv(x, wd, b1, wb)
    gidx = jnp.transpose(idxi, (0, 2, 1)).reshape(Mn)
    vg = _make_sc_gather(C, Mn)(v.reshape(Nn, C), gidx)
    return _edge(u, vg.reshape(Bn, K, P, C), w2, b2, with_out)


def _layer(xs, w1, b1, w2, b2, with_out):
    D = xs[0].shape[-1]
    wd = w1[:D] - w1[D:]
    wb = w1[D:]
    if D == 3:
        xs = [jnp.pad(x, ((0, 0), (0, 0), (0, 5))) for x in xs]
        wd = jnp.pad(wd, ((0, 5), (0, 0)))
        wb = jnp.pad(wb, ((0, 5), (0, 0)))
    if w1.shape[1] == 64:
        wd = jnp.pad(wd, ((0, 0), (0, 64)))
        wb = jnp.pad(wb, ((0, 0), (0, 64)))
        b1 = jnp.pad(b1, (0, 64))
        w2 = jnp.pad(w2, ((0, 64), (0, 0)))
    b1r = b1.reshape(1, -1)
    b2r = b2.reshape(1, -1)
    gather_x = (not with_out) and D % 128 == 0 and D < w1.shape[1]
    return [_half_layer(x, wd, wb, b1r, w2, b2r, with_out, gather_x)
            for x in xs]


_SPLIT = 1


def kernel(pos, batch, w11, b11, w12, b12, w21, b21, w22, b22, w31, b31,
           w32, b32, w41, b41, w42, b42, l1w, l1b, bng, bnb, l2w, l2b):
    x = pos.reshape(B, P, 3)
    g = B // _SPLIT
    xs = [x[i * g:(i + 1) * g] for i in range(_SPLIT)]
    o1 = _layer(xs, w11, b11, w12, b12, True)
    o2 = _layer([o[0] for o in o1], w21, b21, w22, b22, True)
    o3 = _layer([o[0] for o in o2], w31, b31, w32, b32, True)
    o4 = _layer([o[0] for o in o3], w41, b41, w42, b42, False)
    p1 = jnp.concatenate([o[1][:, 0] for o in o1], axis=0)
    p2 = jnp.concatenate([o[1][:, 0] for o in o2], axis=0)
    p3 = jnp.concatenate([o[1][:, 0] for o in o3], axis=0)
    p4 = jnp.concatenate([o[0][:, 0] for o in o4], axis=0)
    return _head(p1, p2, p3, p4, l1w, l1b, bng, bnb, l2w, l2b)


# speedup vs baseline: 1.1000x; 1.1000x over previous
import functools

import jax
import jax.numpy as jnp
from jax import lax
from jax.experimental import pallas as pl
from jax.experimental.pallas import tpu as pltpu
from jax.experimental.pallas import tpu_sc as plsc

B = 16
P = 1024
K = 20
N = B * P
M = K * N

_BLKR = 1024
_BLKE = 512


def _knn_uv_body(x_blk_ref, x_all_ref, wd_ref, b1_ref, wb_ref,
                 idx_ref, u_ref, v_ref):
    b = pl.program_id(0)
    i = pl.program_id(1)
    xb = x_blk_ref[0]
    xa = x_all_ref[0]
    dots = lax.dot_general(xb, xa, (((1,), (1,)), ((), ())),
                           preferred_element_type=jnp.float32)
    sqb = jnp.sum(xb * xb, axis=1, keepdims=True)
    sqa = jnp.sum(xa * xa, axis=1, keepdims=True)
    d2 = jnp.maximum(sqb - 2.0 * dots + sqa.reshape(1, P), 0.0)
    rows = lax.broadcasted_iota(jnp.int32, (_BLKR, P), 0) + i * _BLKR
    cols = lax.broadcasted_iota(jnp.int32, (_BLKR, P), 1)
    ki = lax.bitcast_convert_type(d2, jnp.int32)
    ki = jnp.bitwise_or(jnp.bitwise_and(ki, jnp.int32(~1023)), cols)
    ki = jnp.where(rows == cols, jnp.int32(0x7FFFFFFF), ki)
    base = b * P
    NCH = 2
    H = _BLKR // NCH
    IMIN = jnp.int32(-2147483648)
    kh = tuple(ki[c * H:(c + 1) * H] for c in range(NCH))
    mm = [jnp.full((H, 1), IMIN, jnp.int32) for _ in range(NCH)]
    for t in range(K):
        for s in range(NCH):
            mn = jnp.min(kh[s] - mm[s], axis=1, keepdims=True)
            m = mn + mm[s]
            at = jnp.bitwise_and(m, jnp.int32(1023)) + base
            idx_ref[0, pl.ds(s * H, H), t] = at[:, 0]
            mm[s] = m + (IMIN + 1)
    u_ref[0] = (jnp.dot(xb, wd_ref[...], preferred_element_type=jnp.float32)
                + b1_ref[...])
    if v_ref is not None:
        v_ref[0] = jnp.dot(xb, wb_ref[...],
                           preferred_element_type=jnp.float32)


def _knn_u_body(x_blk_ref, x_all_ref, wd_ref, b1_ref, idx_ref, u_ref):
    _knn_uv_body(x_blk_ref, x_all_ref, wd_ref, b1_ref, None, idx_ref,
                 u_ref, None)


def _knn_uv(x, wd, b1, wb):
    Bn, _, D = x.shape
    C = wd.shape[1]
    grid = (Bn, P // _BLKR)
    return pl.pallas_call(
        _knn_uv_body,
        grid=grid,
        in_specs=[
            pl.BlockSpec((1, _BLKR, D), lambda b, i: (b, i, 0)),
            pl.BlockSpec((1, P, D), lambda b, i: (b, 0, 0)),
            pl.BlockSpec((D, C), lambda b, i: (0, 0)),
            pl.BlockSpec((1, C), lambda b, i: (0, 0)),
            pl.BlockSpec((D, C), lambda b, i: (0, 0)),
        ],
        out_specs=[
            pl.BlockSpec((1, _BLKR, K), lambda b, i: (b, i, 0)),
            pl.BlockSpec((1, _BLKR, C), lambda b, i: (b, i, 0)),
            pl.BlockSpec((1, _BLKR, C), lambda b, i: (b, i, 0)),
        ],
        out_shape=[
            jax.ShapeDtypeStruct((Bn, P, K), jnp.int32),
            jax.ShapeDtypeStruct((Bn, P, C), jnp.float32),
            jax.ShapeDtypeStruct((Bn, P, C), jnp.float32),
        ],
    )(x, x, wd, b1, wb)


def _knn_u(x, wd, b1):
    Bn, _, D = x.shape
    C = wd.shape[1]
    grid = (Bn, P // _BLKR)
    return pl.pallas_call(
        _knn_u_body,
        grid=grid,
        in_specs=[
            pl.BlockSpec((1, _BLKR, D), lambda b, i: (b, i, 0)),
            pl.BlockSpec((1, P, D), lambda b, i: (b, 0, 0)),
            pl.BlockSpec((D, C), lambda b, i: (0, 0)),
            pl.BlockSpec((1, C), lambda b, i: (0, 0)),
        ],
        out_specs=[
            pl.BlockSpec((1, _BLKR, K), lambda b, i: (b, i, 0)),
            pl.BlockSpec((1, _BLKR, C), lambda b, i: (b, i, 0)),
        ],
        out_shape=[
            jax.ShapeDtypeStruct((Bn, P, K), jnp.int32),
            jax.ShapeDtypeStruct((Bn, P, C), jnp.float32),
        ],
    )(x, x, wd, b1)


_SC_INFO = plsc.get_sparse_core_info()
_NC = _SC_INFO.num_cores
_NS = _SC_INFO.num_subcores
_NW = _NC * _NS
_G = 80


@functools.lru_cache(maxsize=None)
def _make_sc_gather(C, Mn):
    per_w = Mn // _NW
    iters = per_w // _G
    mesh = plsc.VectorSubcoreMesh(core_axis_name="c", subcore_axis_name="s")

    @functools.partial(
        pl.kernel, mesh=mesh,
        out_type=jax.ShapeDtypeStruct((Mn, C), jnp.float32),
        scratch_types=[
            pltpu.VMEM((per_w,), jnp.int32),
            pltpu.VMEM((4, _G, C), jnp.float32),
            [pltpu.SemaphoreType.DMA] * 4,
            [pltpu.SemaphoreType.DMA] * 4,
        ],
    )
    def sc_gather(table_hbm, idx_hbm, out_hbm, idxall, rows, gsems, wsems):
        wid = lax.axis_index("s") * _NC + lax.axis_index("c")
        base = wid * per_w
        pltpu.sync_copy(idx_hbm.at[pl.ds(base, per_w)], idxall)

        def start_gather(g, s):
            pltpu.async_copy(
                table_hbm.at[idxall.at[pl.ds(g * _G, _G)]],
                rows.at[s], gsems[s])

        start_gather(0, 0)
        start_gather(1, 1)

        def body(gg, carry):
            for s in range(4):
                g = gg * 4 + s
                s2 = (s + 2) % 4
                pltpu.make_async_copy(
                    table_hbm.at[idxall.at[pl.ds(g * _G, _G)]],
                    rows.at[s], gsems[s]).wait()
                pltpu.async_copy(
                    rows.at[s], out_hbm.at[pl.ds(base + g * _G, _G)],
                    wsems[s])

                @pl.when(jnp.logical_and(g >= 2, g + 2 < iters))
                def _():
                    pltpu.make_async_copy(
                        rows.at[s2], out_hbm.at[pl.ds(base, _G)],
                        wsems[s2]).wait()

                @pl.when(g + 2 < iters)
                def _():
                    start_gather(g + 2, s2)
            return carry

        lax.fori_loop(0, iters // 4, body, 0)
        for s in range(4):
            pltpu.make_async_copy(
                rows.at[s], out_hbm.at[pl.ds(base, _G)], wsems[s]).wait()

    return sc_gather


def _edge_body_full(u_ref, vg_ref, w2_ref, b2_ref, out_ref, pooled_ref):
    i = pl.program_id(1)
    u = u_ref[0]
    acc = jnp.full((u.shape[0], w2_ref.shape[-1]), -1e30, jnp.float32)
    for t in range(K):
        h = jnp.maximum(u + vg_ref[0, t], 0.0)
        acc = jnp.maximum(
            acc, jnp.dot(h, w2_ref[...], preferred_element_type=jnp.float32))
    y = acc + b2_ref[...]
    out_ref[0] = y
    pm = jnp.max(y, axis=0, keepdims=True)

    @pl.when(i == 0)
    def _():
        pooled_ref[0] = pm

    @pl.when(i > 0)
    def _():
        pooled_ref[0] = jnp.maximum(pooled_ref[0], pm)


def _edge_body_pool(u_ref, vg_ref, w2_ref, b2_ref, pooled_ref):
    i = pl.program_id(1)
    u = u_ref[0]
    acc = jnp.full((u.shape[0], w2_ref.shape[-1]), -1e30, jnp.float32)
    for t in range(K):
        h = jnp.maximum(u + vg_ref[0, t], 0.0)
        acc = jnp.maximum(
            acc, jnp.dot(h, w2_ref[...], preferred_element_type=jnp.float32))
    y = acc + b2_ref[...]
    pm = jnp.max(y, axis=0, keepdims=True)

    @pl.when(i == 0)
    def _():
        pooled_ref[0] = pm

    @pl.when(i > 0)
    def _():
        pooled_ref[0] = jnp.maximum(pooled_ref[0], pm)


def _edge(u, vg, w2, b2, with_out):
    Bn, Ci = u.shape[0], u.shape[-1]
    Co = w2.shape[1]
    grid = (Bn, P // _BLKE)
    in_specs = [
        pl.BlockSpec((1, _BLKE, Ci), lambda b, i: (b, i, 0)),
        pl.BlockSpec((1, K, _BLKE, Ci), lambda b, i: (b, 0, i, 0)),
        pl.BlockSpec((Ci, Co), lambda b, i: (0, 0)),
        pl.BlockSpec((1, Co), lambda b, i: (0, 0)),
    ]
    pooled_spec = pl.BlockSpec((1, 1, Co), lambda b, i: (b, 0, 0))
    if with_out:
        return pl.pallas_call(
            _edge_body_full,
            grid=grid,
            in_specs=in_specs,
            out_specs=[pl.BlockSpec((1, _BLKE, Co), lambda b, i: (b, i, 0)),
                       pooled_spec],
            out_shape=[jax.ShapeDtypeStruct((Bn, P, Co), jnp.float32),
                       jax.ShapeDtypeStruct((Bn, 1, Co), jnp.float32)],
        )(u, vg, w2, b2)
    return pl.pallas_call(
        _edge_body_pool,
        grid=grid,
        in_specs=in_specs,
        out_specs=[pooled_spec],
        out_shape=[jax.ShapeDtypeStruct((Bn, 1, Co), jnp.float32)],
    )(u, vg, w2, b2)


def _edge_body_pool_x(u_ref, xg_ref, wb_ref, w2_ref, b2_ref, pooled_ref):
    i = pl.program_id(1)
    u = u_ref[0]
    acc = jnp.full((u.shape[0], w2_ref.shape[-1]), -1e30, jnp.float32)
    for t in range(K):
        vj = jnp.dot(xg_ref[0, t], wb_ref[...],
                     preferred_element_type=jnp.float32)
        h = jnp.maximum(u + vj, 0.0)
        acc = jnp.maximum(
            acc, jnp.dot(h, w2_ref[...], preferred_element_type=jnp.float32))
    y = acc + b2_ref[...]
    pm = jnp.max(y, axis=0, keepdims=True)

    @pl.when(i == 0)
    def _():
        pooled_ref[0] = pm

    @pl.when(i > 0)
    def _():
        pooled_ref[0] = jnp.maximum(pooled_ref[0], pm)


def _edge_x(u, xg, wb, w2, b2):
    Bn, Ci = u.shape[0], u.shape[-1]
    D = xg.shape[-1]
    Co = w2.shape[1]
    grid = (Bn, P // _BLKE)
    return pl.pallas_call(
        _edge_body_pool_x,
        grid=grid,
        in_specs=[
            pl.BlockSpec((1, _BLKE, Ci), lambda b, i: (b, i, 0)),
            pl.BlockSpec((1, K, _BLKE, D), lambda b, i: (b, 0, i, 0)),
            pl.BlockSpec((D, Co), lambda b, i: (0, 0)),
            pl.BlockSpec((Ci, Co), lambda b, i: (0, 0)),
            pl.BlockSpec((1, Co), lambda b, i: (0, 0)),
        ],
        out_specs=[pl.BlockSpec((1, 1, Co), lambda b, i: (b, 0, 0))],
        out_shape=[jax.ShapeDtypeStruct((Bn, 1, Co), jnp.float32)],
    )(u, xg, wb, w2, b2)


def _head_body(p1_ref, p2_ref, p3_ref, p4_ref, wa_ref, wb_ref, wc_ref, wd_ref,
               l1b_ref, bng_ref, bnb_ref, l2w_ref, l2b_ref, out_ref):
    h = (jnp.dot(p1_ref[...], wa_ref[...], preferred_element_type=jnp.float32)
         + jnp.dot(p2_ref[...], wb_ref[...], preferred_element_type=jnp.float32)
         + jnp.dot(p3_ref[...], wc_ref[...], preferred_element_type=jnp.float32)
         + jnp.dot(p4_ref[...], wd_ref[...], preferred_element_type=jnp.float32)
         + l1b_ref[...])
    mu = jnp.mean(h, axis=0, keepdims=True)
    var = jnp.mean((h - mu) * (h - mu), axis=0, keepdims=True)
    hn = (h - mu) / jnp.sqrt(var + 1e-5) * bng_ref[...] + bnb_ref[...]
    hr = jnp.maximum(hn, 0.0)
    logits = (jnp.dot(hr, l2w_ref[...], preferred_element_type=jnp.float32)
              + l2b_ref[...])
    mx = jnp.max(logits, axis=1, keepdims=True)
    z = logits - mx
    lse = jnp.log(jnp.sum(jnp.exp(z), axis=1, keepdims=True))
    out_ref[...] = z - lse


def _head(p1, p2, p3, p4, l1w, l1b, bng, bnb, l2w, l2b):
    wa, wb, wc, wd = l1w[:64], l1w[64:128], l1w[128:256], l1w[256:]
    args = [p1, p2, p3, p4, wa, wb, wc, wd, l1b.reshape(1, -1),
            bng.reshape(1, -1), bnb.reshape(1, -1), l2w, l2b.reshape(1, -1)]
    return pl.pallas_call(
        _head_body,
        out_shape=jax.ShapeDtypeStruct((B, 40), jnp.float32),
    )(*args)


def _half_layer(x, wd, wb, b1, w2, b2, with_out, gather_x):
    Bn, _, D = x.shape
    Nn = Bn * P
    Mn = K * Nn
    C = wd.shape[1]
    if gather_x:
        idxi, u = _knn_u(x, wd, b1)
        gidx = jnp.transpose(idxi, (0, 2, 1)).reshape(Mn)
        xg = _make_sc_gather(D, Mn)(x.reshape(Nn, D), gidx)
        return _edge_x(u, xg.reshape(Bn, K, P, D), wb, w2, b2)
    idxi, u, v = _knn_uv(x, wd, b1, wb)
    gidx = jnp.transpose(idxi, (0, 2, 1)).reshape(Mn)
    vg = _make_sc_gather(C, Mn)(v.reshape(Nn, C), gidx)
    return _edge(u, vg.reshape(Bn, K, P, C), w2, b2, with_out)


def _layer(xs, w1, b1, w2, b2, with_out):
    D = xs[0].shape[-1]
    wd = w1[:D] - w1[D:]
    wb = w1[D:]
    if D == 3:
        xs = [jnp.pad(x, ((0, 0), (0, 0), (0, 5))) for x in xs]
        wd = jnp.pad(wd, ((0, 5), (0, 0)))
        wb = jnp.pad(wb, ((0, 5), (0, 0)))
    if w1.shape[1] == 64:
        wd = jnp.pad(wd, ((0, 0), (0, 64)))
        wb = jnp.pad(wb, ((0, 0), (0, 64)))
        b1 = jnp.pad(b1, (0, 64))
        w2 = jnp.pad(w2, ((0, 64), (0, 0)))
    b1r = b1.reshape(1, -1)
    b2r = b2.reshape(1, -1)
    gather_x = (not with_out) and D % 128 == 0 and D < w1.shape[1]
    return [_half_layer(x, wd, wb, b1r, w2, b2r, with_out, gather_x)
            for x in xs]


_SPLIT = 2


def kernel(pos, batch, w11, b11, w12, b12, w21, b21, w22, b22, w31, b31,
           w32, b32, w41, b41, w42, b42, l1w, l1b, bng, bnb, l2w, l2b):
    x = pos.reshape(B, P, 3)
    g = B // _SPLIT
    xs = [x[i * g:(i + 1) * g] for i in range(_SPLIT)]
    o1 = _layer(xs, w11, b11, w12, b12, True)
    o2 = _layer([o[0] for o in o1], w21, b21, w22, b22, True)
    o3 = _layer([o[0] for o in o2], w31, b31, w32, b32, True)
    o4 = _layer([o[0] for o in o3], w41, b41, w42, b42, False)
    p1 = jnp.concatenate([o[1][:, 0] for o in o1], axis=0)
    p2 = jnp.concatenate([o[1][:, 0] for o in o2], axis=0)
    p3 = jnp.concatenate([o[1][:, 0] for o in o3], axis=0)
    p4 = jnp.concatenate([o[0][:, 0] for o in o4], axis=0)
    return _head(p1, p2, p3, p4, l1w, l1b, bng, bnb, l2w, l2b)


# speedup vs baseline: 1.1138x; 1.0125x over previous
import functools

import jax
import jax.numpy as jnp
from jax import lax
from jax.experimental import pallas as pl
from jax.experimental.pallas import tpu as pltpu
from jax.experimental.pallas import tpu_sc as plsc

B = 16
P = 1024
K = 20
N = B * P
M = K * N

_BLKR = 1024
_BLKE = 512


def _knn_uv_body(x_blk_ref, x_all_ref, wb_ref, idx_ref, v_ref):
    b = pl.program_id(0)
    i = pl.program_id(1)
    xb = x_blk_ref[0]
    xa = x_all_ref[0]
    dots = lax.dot_general(xb, xa, (((1,), (1,)), ((), ())),
                           preferred_element_type=jnp.float32)
    sqb = jnp.sum(xb * xb, axis=1, keepdims=True)
    sqa = jnp.sum(xa * xa, axis=1, keepdims=True)
    d2 = jnp.maximum(sqb - 2.0 * dots + sqa.reshape(1, P), 0.0)
    rows = lax.broadcasted_iota(jnp.int32, (_BLKR, P), 0) + i * _BLKR
    cols = lax.broadcasted_iota(jnp.int32, (_BLKR, P), 1)
    ki = lax.bitcast_convert_type(d2, jnp.int32)
    ki = jnp.bitwise_or(jnp.bitwise_and(ki, jnp.int32(~1023)), cols)
    ki = jnp.where(rows == cols, jnp.int32(0x7FFFFFFF), ki)
    base = b * P
    NCH = 2
    H = _BLKR // NCH
    IMIN = jnp.int32(-2147483648)
    kh = tuple(ki[c * H:(c + 1) * H] for c in range(NCH))
    mm = [jnp.full((H, 1), IMIN, jnp.int32) for _ in range(NCH)]
    for t in range(K):
        for s in range(NCH):
            mn = jnp.min(kh[s] - mm[s], axis=1, keepdims=True)
            m = mn + mm[s]
            at = jnp.bitwise_and(m, jnp.int32(1023)) + base
            idx_ref[0, pl.ds(s * H, H), t] = at[:, 0]
            mm[s] = m + (IMIN + 1)
    if v_ref is not None:
        v_ref[0] = jnp.dot(xb, wb_ref[...],
                           preferred_element_type=jnp.float32)


def _knn_i_body(x_blk_ref, x_all_ref, idx_ref):
    _knn_uv_body(x_blk_ref, x_all_ref, None, idx_ref, None)


def _knn_uv(x, wb):
    Bn, _, D = x.shape
    C = wb.shape[1]
    grid = (Bn, P // _BLKR)
    return pl.pallas_call(
        _knn_uv_body,
        grid=grid,
        in_specs=[
            pl.BlockSpec((1, _BLKR, D), lambda b, i: (b, i, 0)),
            pl.BlockSpec((1, P, D), lambda b, i: (b, 0, 0)),
            pl.BlockSpec((D, C), lambda b, i: (0, 0)),
        ],
        out_specs=[
            pl.BlockSpec((1, _BLKR, K), lambda b, i: (b, i, 0)),
            pl.BlockSpec((1, _BLKR, C), lambda b, i: (b, i, 0)),
        ],
        out_shape=[
            jax.ShapeDtypeStruct((Bn, P, K), jnp.int32),
            jax.ShapeDtypeStruct((Bn, P, C), jnp.float32),
        ],
    )(x, x, wb)


def _knn_i(x):
    Bn, _, D = x.shape
    grid = (Bn, P // _BLKR)
    return pl.pallas_call(
        _knn_i_body,
        grid=grid,
        in_specs=[
            pl.BlockSpec((1, _BLKR, D), lambda b, i: (b, i, 0)),
            pl.BlockSpec((1, P, D), lambda b, i: (b, 0, 0)),
        ],
        out_specs=[
            pl.BlockSpec((1, _BLKR, K), lambda b, i: (b, i, 0)),
        ],
        out_shape=[
            jax.ShapeDtypeStruct((Bn, P, K), jnp.int32),
        ],
    )(x, x)


_SC_INFO = plsc.get_sparse_core_info()
_NC = _SC_INFO.num_cores
_NS = _SC_INFO.num_subcores
_NW = _NC * _NS
_G = 80


@functools.lru_cache(maxsize=None)
def _make_sc_gather(C, Mn):
    per_w = Mn // _NW
    iters = per_w // _G
    mesh = plsc.VectorSubcoreMesh(core_axis_name="c", subcore_axis_name="s")

    @functools.partial(
        pl.kernel, mesh=mesh,
        out_type=jax.ShapeDtypeStruct((Mn, C), jnp.float32),
        scratch_types=[
            pltpu.VMEM((per_w,), jnp.int32),
            pltpu.VMEM((4, _G, C), jnp.float32),
            [pltpu.SemaphoreType.DMA] * 4,
            [pltpu.SemaphoreType.DMA] * 4,
        ],
    )
    def sc_gather(table_hbm, idx_hbm, out_hbm, idxall, rows, gsems, wsems):
        wid = lax.axis_index("s") * _NC + lax.axis_index("c")
        base = wid * per_w
        pltpu.sync_copy(idx_hbm.at[pl.ds(base, per_w)], idxall)

        def start_gather(g, s):
            pltpu.async_copy(
                table_hbm.at[idxall.at[pl.ds(g * _G, _G)]],
                rows.at[s], gsems[s])

        start_gather(0, 0)
        start_gather(1, 1)

        def body(gg, carry):
            for s in range(4):
                g = gg * 4 + s
                s2 = (s + 2) % 4
                pltpu.make_async_copy(
                    table_hbm.at[idxall.at[pl.ds(g * _G, _G)]],
                    rows.at[s], gsems[s]).wait()
                pltpu.async_copy(
                    rows.at[s], out_hbm.at[pl.ds(base + g * _G, _G)],
                    wsems[s])

                @pl.when(jnp.logical_and(g >= 2, g + 2 < iters))
                def _():
                    pltpu.make_async_copy(
                        rows.at[s2], out_hbm.at[pl.ds(base, _G)],
                        wsems[s2]).wait()

                @pl.when(g + 2 < iters)
                def _():
                    start_gather(g + 2, s2)
            return carry

        lax.fori_loop(0, iters // 4, body, 0)
        for s in range(4):
            pltpu.make_async_copy(
                rows.at[s], out_hbm.at[pl.ds(base, _G)], wsems[s]).wait()

    return sc_gather


def _edge_body_full(x_ref, vg_ref, wd_ref, b1_ref, w2_ref, b2_ref,
                    out_ref, pooled_ref):
    i = pl.program_id(1)
    u = (jnp.dot(x_ref[0], wd_ref[...], preferred_element_type=jnp.float32)
         + b1_ref[...])
    acc = jnp.full((u.shape[0], w2_ref.shape[-1]), -1e30, jnp.float32)
    for t in range(K):
        h = jnp.maximum(u + vg_ref[0, t], 0.0)
        acc = jnp.maximum(
            acc, jnp.dot(h, w2_ref[...], preferred_element_type=jnp.float32))
    y = acc + b2_ref[...]
    out_ref[0] = y
    pm = jnp.max(y, axis=0, keepdims=True)

    @pl.when(i == 0)
    def _():
        pooled_ref[0] = pm

    @pl.when(i > 0)
    def _():
        pooled_ref[0] = jnp.maximum(pooled_ref[0], pm)


def _edge_body_pool(x_ref, vg_ref, wd_ref, b1_ref, w2_ref, b2_ref,
                    pooled_ref):
    i = pl.program_id(1)
    u = (jnp.dot(x_ref[0], wd_ref[...], preferred_element_type=jnp.float32)
         + b1_ref[...])
    acc = jnp.full((u.shape[0], w2_ref.shape[-1]), -1e30, jnp.float32)
    for t in range(K):
        h = jnp.maximum(u + vg_ref[0, t], 0.0)
        acc = jnp.maximum(
            acc, jnp.dot(h, w2_ref[...], preferred_element_type=jnp.float32))
    y = acc + b2_ref[...]
    pm = jnp.max(y, axis=0, keepdims=True)

    @pl.when(i == 0)
    def _():
        pooled_ref[0] = pm

    @pl.when(i > 0)
    def _():
        pooled_ref[0] = jnp.maximum(pooled_ref[0], pm)


def _edge(x, vg, wd, b1, w2, b2, with_out):
    Bn, D = x.shape[0], x.shape[-1]
    Ci = wd.shape[1]
    Co = w2.shape[1]
    grid = (Bn, P // _BLKE)
    in_specs = [
        pl.BlockSpec((1, _BLKE, D), lambda b, i: (b, i, 0)),
        pl.BlockSpec((1, K, _BLKE, Ci), lambda b, i: (b, 0, i, 0)),
        pl.BlockSpec((D, Ci), lambda b, i: (0, 0)),
        pl.BlockSpec((1, Ci), lambda b, i: (0, 0)),
        pl.BlockSpec((Ci, Co), lambda b, i: (0, 0)),
        pl.BlockSpec((1, Co), lambda b, i: (0, 0)),
    ]
    pooled_spec = pl.BlockSpec((1, 1, Co), lambda b, i: (b, 0, 0))
    if with_out:
        return pl.pallas_call(
            _edge_body_full,
            grid=grid,
            in_specs=in_specs,
            out_specs=[pl.BlockSpec((1, _BLKE, Co), lambda b, i: (b, i, 0)),
                       pooled_spec],
            out_shape=[jax.ShapeDtypeStruct((Bn, P, Co), jnp.float32),
                       jax.ShapeDtypeStruct((Bn, 1, Co), jnp.float32)],
        )(x, vg, wd, b1, w2, b2)
    return pl.pallas_call(
        _edge_body_pool,
        grid=grid,
        in_specs=in_specs,
        out_specs=[pooled_spec],
        out_shape=[jax.ShapeDtypeStruct((Bn, 1, Co), jnp.float32)],
    )(x, vg, wd, b1, w2, b2)


def _edge_body_pool_x(x_ref, xg_ref, wd_ref, b1_ref, wb_ref, w2_ref, b2_ref,
                      pooled_ref):
    i = pl.program_id(1)
    u = (jnp.dot(x_ref[0], wd_ref[...], preferred_element_type=jnp.float32)
         + b1_ref[...])
    acc = jnp.full((u.shape[0], w2_ref.shape[-1]), -1e30, jnp.float32)
    for t in range(K):
        vj = jnp.dot(xg_ref[0, t], wb_ref[...],
                     preferred_element_type=jnp.float32)
        h = jnp.maximum(u + vj, 0.0)
        acc = jnp.maximum(
            acc, jnp.dot(h, w2_ref[...], preferred_element_type=jnp.float32))
    y = acc + b2_ref[...]
    pm = jnp.max(y, axis=0, keepdims=True)

    @pl.when(i == 0)
    def _():
        pooled_ref[0] = pm

    @pl.when(i > 0)
    def _():
        pooled_ref[0] = jnp.maximum(pooled_ref[0], pm)


def _edge_x(x, xg, wd, b1, wb, w2, b2):
    Bn, D = x.shape[0], x.shape[-1]
    Ci = wd.shape[1]
    Co = w2.shape[1]
    grid = (Bn, P // _BLKE)
    return pl.pallas_call(
        _edge_body_pool_x,
        grid=grid,
        in_specs=[
            pl.BlockSpec((1, _BLKE, D), lambda b, i: (b, i, 0)),
            pl.BlockSpec((1, K, _BLKE, D), lambda b, i: (b, 0, i, 0)),
            pl.BlockSpec((D, Ci), lambda b, i: (0, 0)),
            pl.BlockSpec((1, Ci), lambda b, i: (0, 0)),
            pl.BlockSpec((D, Co), lambda b, i: (0, 0)),
            pl.BlockSpec((Ci, Co), lambda b, i: (0, 0)),
            pl.BlockSpec((1, Co), lambda b, i: (0, 0)),
        ],
        out_specs=[pl.BlockSpec((1, 1, Co), lambda b, i: (b, 0, 0))],
        out_shape=[jax.ShapeDtypeStruct((Bn, 1, Co), jnp.float32)],
    )(x, xg, wd, b1, wb, w2, b2)


def _head_body(p1_ref, p2_ref, p3_ref, p4_ref, wa_ref, wb_ref, wc_ref, wd_ref,
               l1b_ref, bng_ref, bnb_ref, l2w_ref, l2b_ref, out_ref):
    h = (jnp.dot(p1_ref[...], wa_ref[...], preferred_element_type=jnp.float32)
         + jnp.dot(p2_ref[...], wb_ref[...], preferred_element_type=jnp.float32)
         + jnp.dot(p3_ref[...], wc_ref[...], preferred_element_type=jnp.float32)
         + jnp.dot(p4_ref[...], wd_ref[...], preferred_element_type=jnp.float32)
         + l1b_ref[...])
    mu = jnp.mean(h, axis=0, keepdims=True)
    var = jnp.mean((h - mu) * (h - mu), axis=0, keepdims=True)
    hn = (h - mu) / jnp.sqrt(var + 1e-5) * bng_ref[...] + bnb_ref[...]
    hr = jnp.maximum(hn, 0.0)
    logits = (jnp.dot(hr, l2w_ref[...], preferred_element_type=jnp.float32)
              + l2b_ref[...])
    mx = jnp.max(logits, axis=1, keepdims=True)
    z = logits - mx
    lse = jnp.log(jnp.sum(jnp.exp(z), axis=1, keepdims=True))
    out_ref[...] = z - lse


def _head(p1, p2, p3, p4, l1w, l1b, bng, bnb, l2w, l2b):
    wa, wb, wc, wd = l1w[:64], l1w[64:128], l1w[128:256], l1w[256:]
    args = [p1, p2, p3, p4, wa, wb, wc, wd, l1b.reshape(1, -1),
            bng.reshape(1, -1), bnb.reshape(1, -1), l2w, l2b.reshape(1, -1)]
    return pl.pallas_call(
        _head_body,
        out_shape=jax.ShapeDtypeStruct((B, 40), jnp.float32),
    )(*args)


def _half_layer(x, wd, wb, b1, w2, b2, with_out, gather_x):
    Bn, _, D = x.shape
    Nn = Bn * P
    Mn = K * Nn
    C = wd.shape[1]
    if gather_x:
        (idxi,) = _knn_i(x)
        gidx = jnp.transpose(idxi, (0, 2, 1)).reshape(Mn)
        xg = _make_sc_gather(x.shape[-1], Mn)(x.reshape(Nn, x.shape[-1]),
                                              gidx)
        return _edge_x(x, xg.reshape(Bn, K, P, x.shape[-1]), wd, b1, wb,
                       w2, b2)
    idxi, v = _knn_uv(x, wb)
    gidx = jnp.transpose(idxi, (0, 2, 1)).reshape(Mn)
    vg = _make_sc_gather(C, Mn)(v.reshape(Nn, C), gidx)
    return _edge(x, vg.reshape(Bn, K, P, C), wd, b1, w2, b2, with_out)


def _layer(xs, w1, b1, w2, b2, with_out):
    D = xs[0].shape[-1]
    wd = w1[:D] - w1[D:]
    wb = w1[D:]
    if D == 3:
        xs = [jnp.pad(x, ((0, 0), (0, 0), (0, 5))) for x in xs]
        wd = jnp.pad(wd, ((0, 5), (0, 0)))
        wb = jnp.pad(wb, ((0, 5), (0, 0)))
    if w1.shape[1] == 64:
        wd = jnp.pad(wd, ((0, 0), (0, 64)))
        wb = jnp.pad(wb, ((0, 0), (0, 64)))
        b1 = jnp.pad(b1, (0, 64))
        w2 = jnp.pad(w2, ((0, 64), (0, 0)))
    b1r = b1.reshape(1, -1)
    b2r = b2.reshape(1, -1)
    gather_x = (not with_out) and D % 128 == 0 and D < w1.shape[1]
    return [_half_layer(x, wd, wb, b1r, w2, b2r, with_out, gather_x)
            for x in xs]


_SPLIT = 2


def kernel(pos, batch, w11, b11, w12, b12, w21, b21, w22, b22, w31, b31,
           w32, b32, w41, b41, w42, b42, l1w, l1b, bng, bnb, l2w, l2b):
    x = pos.reshape(B, P, 3)
    g = B // _SPLIT
    xs = [x[i * g:(i + 1) * g] for i in range(_SPLIT)]
    o1 = _layer(xs, w11, b11, w12, b12, True)
    o2 = _layer([o[0] for o in o1], w21, b21, w22, b22, True)
    o3 = _layer([o[0] for o in o2], w31, b31, w32, b32, True)
    o4 = _layer([o[0] for o in o3], w41, b41, w42, b42, False)
    p1 = jnp.concatenate([o[1][:, 0] for o in o1], axis=0)
    p2 = jnp.concatenate([o[1][:, 0] for o in o2], axis=0)
    p3 = jnp.concatenate([o[1][:, 0] for o in o3], axis=0)
    p4 = jnp.concatenate([o[0][:, 0] for o in o4], axis=0)
    return _head(p1, p2, p3, p4, l1w, l1b, bng, bnb, l2w, l2b)
